# Initial kernel scaffold; baseline (speedup 1.0000x reference)
import jax
import jax.numpy as jnp
from jax.experimental import pallas as pl


def kernel(x_user, x_item, edge_rates, edge_rated_by, Wsrc_r, Wdst_r, attn_r, bias_r, Wsrc_rb, Wdst_rb, attn_rb, bias_rb, bn_gamma, bn_beta):
    raise NotImplementedError("write your pallas kernel here")


# speedup vs baseline: 14.7905x; 14.7905x over previous
import jax
import jax.numpy as jnp
import numpy as np
from jax import lax
from jax.experimental import pallas as pl
from jax.experimental.pallas import tpu as pltpu
from jax.experimental.pallas import tpu_sc as plsc

N = 10000
E = 320000
D = 128
H = 8
DH = 16

NP = 10240
C = 80
NW = 32
EW = E // NW
NCH = EW // C
STRIPE = NP // 16

_I16 = np.arange(16)
_PERMS = np.stack([_I16 ^ 8, _I16 ^ 4, _I16 ^ 2, _I16 ^ 1]).astype(np.int32)
_EYE = np.eye(H, 16, dtype=np.float32)


def _proj_body(xu, xi, w1, w2, w3, w4, o1, o2, o3, o4):
    o1[...] = jnp.dot(xu[...], w1[...], preferred_element_type=jnp.float32)
    o2[...] = jnp.dot(xi[...], w2[...], preferred_element_type=jnp.float32)
    o3[...] = jnp.dot(xi[...], w3[...], preferred_element_type=jnp.float32)
    o4[...] = jnp.dot(xu[...], w4[...], preferred_element_type=jnp.float32)


def _project(x_user, x_item, Wsrc_r, Wdst_r, Wsrc_rb, Wdst_rb):
    blk = 2000
    grid = (N // blk,)
    row = pl.BlockSpec((blk, D), lambda b: (b, 0))
    full = pl.BlockSpec((D, D), lambda b: (0, 0))
    outs = [jax.ShapeDtypeStruct((N, D), jnp.float32)] * 4
    return pl.pallas_call(
        _proj_body,
        grid=grid,
        in_specs=[row, row, full, full, full, full],
        out_specs=[row, row, row, row],
        out_shape=outs,
    )(x_user, x_item, Wsrc_r, Wdst_r, Wsrc_rb, Wdst_rb)


def _sc_body(fs0_hbm, fd0_hbm, e0_hbm, attn0_hbm,
             fs1_hbm, fd1_hbm, e1_hbm, attn1_hbm,
             perm_hbm, eye_hbm,
             onum_hbm, oden_hbm, wbuf_hbm,
             sidx, didx, fsb, fdb, cden, attnb, permb, eyeb,
             anum, sem1, sem2):
    c = lax.axis_index("c")
    s = lax.axis_index("s")
    wid = c * 16 + s

    pltpu.sync_copy(perm_hbm, permb)
    pltpu.sync_copy(eye_hbm, eyeb)

    rels = ((fs0_hbm, fd0_hbm, e0_hbm, attn0_hbm),
            (fs1_hbm, fd1_hbm, e1_hbm, attn1_hbm))
    for rel in range(2):
        fs_hbm, fd_hbm, e_hbm, attn_hbm = rels[rel]
        for phase in range(2):
            def zrow(r, carry):
                zv = jnp.zeros((16,), jnp.float32)
                for j in range(D // 16):
                    fsb[r, pl.ds(16 * j, 16)] = zv
                return carry

            lax.fori_loop(0, C, zrow, 0)

            def zacc(j, carry):
                pltpu.sync_copy(fsb, anum.at[pl.ds(s * STRIPE + j * C, C)])
                return carry

            lax.fori_loop(0, STRIPE // C, zacc, 0)
            if phase == 0:
                pltpu.sync_copy(attn_hbm, attnb)
            plsc.subcore_barrier()

            if phase == 0:
                def chunk0(k, carry):
                    base = wid * EW + k * C
                    pltpu.sync_copy(e_hbm.at[pl.ds(base, C)], sidx)
                    pltpu.sync_copy(e_hbm.at[pl.ds(E + base, C)], didx)
                    cp1 = pltpu.async_copy(fs_hbm.at[sidx], fsb, sem1)
                    cp2 = pltpu.async_copy(fd_hbm.at[didx], fdb, sem2)
                    cp1.wait()
                    cp2.wait()

                    def edge(i, ecarry):
                        denrow = jnp.zeros((16,), jnp.float32)
                        for h in range(H):
                            f = fsb[i, pl.ds(16 * h, 16)]
                            t = f + fdb[i, pl.ds(16 * h, 16)]
                            t = jnp.maximum(t, t * 0.2)
                            q = t * attnb[h, :]
                            for st in range(4):
                                q = q + q.at[permb[st, :]].get(
                                    mode="promise_in_bounds")
                            wv = jnp.exp(q)
                            fsb[i, pl.ds(16 * h, 16)] = wv * f
                            denrow = denrow + wv * eyeb[h, :]
                        cden[i, :] = denrow
                        return ecarry

                    lax.fori_loop(0, C, edge, 0)
                    pltpu.sync_copy(fsb, anum.at[didx], add=True)
                    pltpu.sync_copy(cden, wbuf_hbm.at[pl.ds(rel * E + base, C)])
                    return carry

                lax.fori_loop(0, NCH, chunk0, 0)
            else:
                def chunk1(k, carry):
                    base = wid * EW + k * C
                    pltpu.sync_copy(e_hbm.at[pl.ds(E + base, C)], didx)
                    pltpu.sync_copy(wbuf_hbm.at[pl.ds(rel * E + base, C)], cden)

                    def edge(i, ecarry):
                        wrow = cden[i, :]
                        for h in range(H):
                            fsb[i, pl.ds(16 * h, 16)] = wrow * eyeb[h, :]
                        return ecarry

                    lax.fori_loop(0, C, edge, 0)
                    pltpu.sync_copy(fsb, anum.at[didx], add=True)
                    return carry

                lax.fori_loop(0, NCH, chunk1, 0)

            plsc.subcore_barrier()
            out_hbm = onum_hbm if phase == 0 else oden_hbm

            def cpout(j, carry):
                r0 = s * STRIPE + j * C
                pltpu.sync_copy(anum.at[pl.ds(r0, C)], fsb)
                pltpu.sync_copy(
                    fsb, out_hbm.at[pl.ds(rel * 2 * NP + c * NP + r0, C)])
                return carry

            lax.fori_loop(0, STRIPE // C, cpout, 0)
            plsc.subcore_barrier()


def _sc_edge2(fs0, fd0, e0, attn0, fs1, fd1, e1, attn1):
    mesh = plsc.VectorSubcoreMesh(core_axis_name="c", subcore_axis_name="s")
    kfn = pl.kernel(
        _sc_body,
        out_type=(jax.ShapeDtypeStruct((4 * NP, D), jnp.float32),
                  jax.ShapeDtypeStruct((4 * NP, D), jnp.float32),
                  jax.ShapeDtypeStruct((2 * E, DH), jnp.float32)),
        mesh=mesh,
        scratch_types=[
            pltpu.VMEM((C,), jnp.int32),
            pltpu.VMEM((C,), jnp.int32),
            pltpu.VMEM((C, D), jnp.float32),
            pltpu.VMEM((C, D), jnp.float32),
            pltpu.VMEM((C, DH), jnp.float32),
            pltpu.VMEM((H, DH), jnp.float32),
            pltpu.VMEM((4, 16), jnp.int32),
            pltpu.VMEM((H, 16), jnp.float32),
            pltpu.VMEM_SHARED((NP, D), jnp.float32),
            pltpu.SemaphoreType.DMA,
            pltpu.SemaphoreType.DMA,
        ],
    )
    return kfn(fs0, fd0, e0, attn0, fs1, fd1, e1, attn1,
               jnp.asarray(_PERMS), jnp.asarray(_EYE))


def _fin_body(p_ref, pd_ref, b_ref, g_ref, bt_ref, o_ref):
    p = p_ref[0]
    num = p[:NP][:N] + p[NP:][:N]
    pd = pd_ref[0]
    dsum = pd[:NP][:N] + pd[NP:][:N]
    col = lax.broadcasted_iota(jnp.int32, (D, D), 1)
    row = lax.broadcasted_iota(jnp.int32, (D, D), 0)
    rep = (row == (col // DH) * (DH + 1)).astype(jnp.float32)
    den16 = jnp.dot(dsum, rep, preferred_element_type=jnp.float32)
    hpre = num / (den16 + 1e-9) + b_ref[0]
    mu = jnp.mean(hpre, axis=0, keepdims=True)
    var = jnp.mean(hpre * hpre, axis=0, keepdims=True) - mu * mu
    y = g_ref[...] * (hpre - mu) * lax.rsqrt(var + 1e-5) + bt_ref[...]
    o_ref[0] = jnp.maximum(y, 0.01 * y)


def _finalize(onum, oden, bias_rb, bias_r, bn_gamma, bn_beta):
    P = onum.reshape(2, 2 * NP, D)
    PD = oden.reshape(2, 2 * NP, D)
    B = jnp.stack([bias_rb, bias_r]).reshape(2, 1, D)
    out = pl.pallas_call(
        _fin_body,
        grid=(2,),
        in_specs=[
            pl.BlockSpec((1, 2 * NP, D), lambda r: (1 - r, 0, 0)),
            pl.BlockSpec((1, 2 * NP, D), lambda r: (1 - r, 0, 0)),
            pl.BlockSpec((1, 1, D), lambda r: (r, 0, 0)),
            pl.BlockSpec((1, D), lambda r: (0, 0)),
            pl.BlockSpec((1, D), lambda r: (0, 0)),
        ],
        out_specs=pl.BlockSpec((1, N, D), lambda r: (r, 0, 0)),
        out_shape=jax.ShapeDtypeStruct((2, N, D), jnp.float32),
    )(P, PD, B, bn_gamma.reshape(1, D), bn_beta.reshape(1, D))
    return out[0], out[1]


def kernel(x_user, x_item, edge_rates, edge_rated_by,
           Wsrc_r, Wdst_r, attn_r, bias_r,
           Wsrc_rb, Wdst_rb, attn_rb, bias_rb,
           bn_gamma, bn_beta):
    e_r = edge_rates.astype(jnp.int32).reshape(2 * E)
    e_rb = edge_rated_by.astype(jnp.int32).reshape(2 * E)

    fs_r, fd_r, fs_rb, fd_rb = _project(
        x_user, x_item, Wsrc_r, Wdst_r, Wsrc_rb, Wdst_rb)

    onum, oden, _ = _sc_edge2(fs_r, fd_r, e_r, attn_r,
                              fs_rb, fd_rb, e_rb, attn_rb)

    h_user, h_item = _finalize(onum, oden, bias_rb, bias_r,
                               bn_gamma, bn_beta)
    return (h_user, h_item)
